# baseline (speedup 1.0000x reference)
import functools

import jax
import jax.numpy as jnp
from jax import lax
from jax.experimental import pallas as pl
from jax.experimental.pallas import tpu as pltpu
from jax.experimental.pallas import tpu_sc as plsc

N = 10000
E = 320000
DIN = 128
DHID = 256
DOUT = 64

NC = 2
NS = 16
NW = NC * NS
G = 64
NF = 160
EPW = NF * G
EPAD = NW * EPW
NPAD = 10240
RPT = NPAD // NS


NB = 4


def _make_agg(D, dt):
  mesh = plsc.VectorSubcoreMesh(core_axis_name="c", subcore_axis_name="s")
  NFH = NF // 2
  EPH = NFH * G
  assert NFH % NB == 0 and NFH // NB >= 2

  @functools.partial(
      pl.kernel,
      out_type=jax.ShapeDtypeStruct((NC, NPAD, D), dt),
      mesh=mesh,
      scratch_types=(
          [pltpu.VMEM((EPH,), jnp.int32)] * 2
          + [pltpu.VMEM((G, D), dt)] * NB
          + [pltpu.VMEM_SHARED((NPAD, D), dt)]
          + [pltpu.SemaphoreType.DMA] * (2 * NB)
      ),
  )
  def agg(src_hbm, dst_hbm, rows_hbm, zeros_hbm, out_hbm, *scr):
    src_v, dst_v = scr[0], scr[1]
    bufs = scr[2:2 + NB]
    acc = scr[2 + NB]
    gsem = scr[3 + NB:3 + 2 * NB]
    ssem = scr[3 + 2 * NB:3 + 3 * NB]
    c = lax.axis_index("c")
    s = lax.axis_index("s")
    wid = c * NS + s
    pltpu.sync_copy(zeros_hbm, acc.at[pl.ds(s * RPT, RPT)])
    plsc.subcore_barrier()

    def gather(j, b):
      pltpu.async_copy(rows_hbm.at[src_v.at[pl.ds(j * G, G)]], bufs[b],
                       gsem[b])

    def gwait(j, b):
      pltpu.make_async_copy(rows_hbm.at[src_v.at[pl.ds(j * G, G)]], bufs[b],
                            gsem[b]).wait()

    def scat(j, b):
      pltpu.async_copy(bufs[b], acc.at[dst_v.at[pl.ds(j * G, G)]], ssem[b],
                       add=True)

    def swait(j, b):
      pltpu.make_async_copy(bufs[b], acc.at[dst_v.at[pl.ds(j * G, G)]],
                            ssem[b]).wait()

    def step(j, b, has_prev, has_next):
      gwait(j, b)
      scat(j, b)
      if has_prev:
        swait(j - 1, (b - 1) % NB)
      if has_next:
        gather(j + NB - 1, (b - 1) % NB)

    for h in range(2):
      pltpu.sync_copy(src_hbm.at[wid, pl.ds(h * EPH, EPH)], src_v)
      pltpu.sync_copy(dst_hbm.at[wid, pl.ds(h * EPH, EPH)], dst_v)
      for j in range(NB - 1):
        gather(j, j)
      for i in range(NB):
        step(i, i, i > 0, True)

      def body(g, carry):
        for i in range(NB):
          step(g * NB + i, i, True, True)
        return carry

      lax.fori_loop(1, NFH // NB - 1, body, 0)
      for i in range(NB):
        j = NFH - NB + i
        step(j, i, True, j + NB - 1 < NFH)
      swait(NFH - 1, NB - 1)
    plsc.subcore_barrier()
    pltpu.sync_copy(acc.at[pl.ds(s * RPT, RPT)],
                    out_hbm.at[c, pl.ds(s * RPT, RPT)])

  return agg


_agg128 = _make_agg(DIN, jnp.float32)


def _make_deg():
  mesh = plsc.VectorSubcoreMesh(core_axis_name="c", subcore_axis_name="s")

  @functools.partial(
      pl.kernel,
      out_type=jax.ShapeDtypeStruct((NC, 1, NPAD), jnp.float32),
      mesh=mesh,
      scratch_types=[
          pltpu.VMEM((NF, G), jnp.int32),
          pltpu.VMEM((G,), jnp.float32),
          pltpu.VMEM((RPT,), jnp.float32),
          pltpu.VMEM_SHARED((NPAD,), jnp.float32),
      ],
  )
  def deg(dst_hbm, ones_hbm, zeros_hbm, out_hbm,
          dst_v, ones_v, zeros_v, acc):
    c = lax.axis_index("c")
    s = lax.axis_index("s")
    wid = c * NS + s
    pltpu.sync_copy(dst_hbm.at[wid], dst_v)
    pltpu.sync_copy(ones_hbm, ones_v)
    pltpu.sync_copy(zeros_hbm, zeros_v)
    pltpu.sync_copy(zeros_v, acc.at[pl.ds(s * RPT, RPT)])
    plsc.subcore_barrier()

    def body(j, carry):
      pltpu.sync_copy(ones_v, acc.at[dst_v.at[j]], add=True)
      return carry

    lax.fori_loop(0, NF, body, 0)
    plsc.subcore_barrier()
    pltpu.sync_copy(acc.at[pl.ds(s * RPT, RPT)],
                    out_hbm.at[c, 0, pl.ds(s * RPT, RPT)])

  return deg


_deg = _make_deg()

_B = 2048


def _dinv_of(degp_ref):
  dp = degp_ref[0, 0, :] + degp_ref[1, 0, :]
  return lax.rsqrt(dp + 1.0)[:, None]


def _tc1_body(degp_ref, x_ref, xs_ref):
  xs_ref[...] = x_ref[...] * _dinv_of(degp_ref)


def _tc2_body(degp_ref, aggp_ref, xs_ref, w1_ref, b1_ref, w2_ref, t_ref):
  dinv = _dinv_of(degp_ref)
  agg = (aggp_ref[0] + aggp_ref[1] + xs_ref[...]) * dinv
  h = jnp.dot(agg, w1_ref[...], preferred_element_type=jnp.float32,
              precision=lax.Precision.HIGHEST) + b1_ref[...]
  h = jnp.maximum(h, 0.0)
  t = jnp.dot(h, w2_ref[...], preferred_element_type=jnp.float32,
              precision=lax.Precision.HIGHEST) * dinv
  t_ref[...] = jnp.concatenate([t, jnp.zeros_like(t)], axis=1)


def _tc3_body(degp_ref, aggp_ref, t_ref, b2_ref, o_ref):
  agg = (aggp_ref[0] + aggp_ref[1] + t_ref[...])[:, :DOUT]
  o_ref[...] = agg * _dinv_of(degp_ref) + b2_ref[...]


def _degp_spec():
  return pl.BlockSpec((NC, 1, _B), lambda i: (0, 0, i))


def _rows_spec(d):
  return pl.BlockSpec((_B, d), lambda i: (i, 0))


def _aggp_spec(d):
  return pl.BlockSpec((NC, _B, d), lambda i: (0, i, 0))


def _full_spec(shape):
  return pl.BlockSpec(shape, lambda i: tuple(0 for _ in shape))


_tc1 = pl.pallas_call(
    _tc1_body,
    grid=(NPAD // _B,),
    in_specs=[_degp_spec(), _rows_spec(DIN)],
    out_specs=_rows_spec(DIN),
    out_shape=jax.ShapeDtypeStruct((NPAD, DIN), jnp.float32),
)

_tc2 = pl.pallas_call(
    _tc2_body,
    grid=(NPAD // _B,),
    in_specs=[_degp_spec(), _aggp_spec(DIN), _rows_spec(DIN),
              _full_spec((DIN, DHID)), _full_spec((1, DHID)),
              _full_spec((DHID, DOUT))],
    out_specs=_rows_spec(DIN),
    out_shape=jax.ShapeDtypeStruct((NPAD, DIN), jnp.float32),
)

_tc3 = pl.pallas_call(
    _tc3_body,
    grid=(NPAD // _B,),
    in_specs=[_degp_spec(), _aggp_spec(DIN), _rows_spec(DIN),
              _full_spec((1, DOUT))],
    out_specs=_rows_spec(DOUT),
    out_shape=jax.ShapeDtypeStruct((NPAD, DOUT), jnp.float32),
)


def kernel(x, edge_index, W1, b1, W2, b2):
  ei = edge_index.astype(jnp.int32)
  npad_e = EPAD - E
  pad_src = jnp.arange(npad_e, dtype=jnp.int32) % N
  pad_dst = N + jnp.arange(npad_e, dtype=jnp.int32) % (NPAD - N)
  src = jnp.concatenate([ei[0], pad_src]).reshape(NW, EPW)
  dst = jnp.concatenate([ei[1], pad_dst]).reshape(NW, EPW)
  zeros_in = jnp.zeros((RPT, DIN), jnp.float32)
  ones_1d = jnp.ones((G,), jnp.float32)
  zeros_1d = jnp.zeros((RPT,), jnp.float32)

  xp = jnp.pad(x, ((0, NPAD - N), (0, 0)))
  degp = _deg(dst.reshape(NW, NF, G), ones_1d, zeros_1d)
  xs = _tc1(degp, xp)
  aggp1 = _agg128(src, dst, xs, zeros_in)
  t = _tc2(degp, aggp1, xs, W1, b1.reshape(1, DHID), W2)
  aggp2 = _agg128(src, dst, t, zeros_in)
  return _tc3(degp, aggp2, t, b2.reshape(1, DOUT))[:N]


# speedup vs baseline: 1.0124x; 1.0124x over previous
import functools

import jax
import jax.numpy as jnp
from jax import lax
from jax.experimental import pallas as pl
from jax.experimental.pallas import tpu as pltpu
from jax.experimental.pallas import tpu_sc as plsc

N = 10000
E = 320000
DIN = 128
DHID = 256
DOUT = 64

NC = 2
NS = 16
NW = NC * NS
G = 64
NF = 160
EPW = NF * G
EPAD = NW * EPW
NPAD = 10240
RPT = NPAD // NS


NB = 5
NST = 4


def _make_agg(D, dt):
  mesh = plsc.VectorSubcoreMesh(core_axis_name="c", subcore_axis_name="s")
  NFH = NF // NST
  EPH = NFH * G
  assert NFH % NB == 0 and NFH // NB >= 2

  @functools.partial(
      pl.kernel,
      out_type=jax.ShapeDtypeStruct((NC, NPAD, D), dt),
      mesh=mesh,
      scratch_types=(
          [pltpu.VMEM((EPH,), jnp.int32)] * 2
          + [pltpu.VMEM((G, D), dt)] * NB
          + [pltpu.VMEM_SHARED((NPAD, D), dt)]
          + [pltpu.SemaphoreType.DMA] * (2 * NB)
      ),
  )
  def agg(src_hbm, dst_hbm, rows_hbm, zeros_hbm, out_hbm, *scr):
    src_v, dst_v = scr[0], scr[1]
    bufs = scr[2:2 + NB]
    acc = scr[2 + NB]
    gsem = scr[3 + NB:3 + 2 * NB]
    ssem = scr[3 + 2 * NB:3 + 3 * NB]
    c = lax.axis_index("c")
    s = lax.axis_index("s")
    wid = c * NS + s
    pltpu.sync_copy(zeros_hbm, acc.at[pl.ds(s * RPT, RPT)])
    plsc.subcore_barrier()

    def gather(j, b):
      pltpu.async_copy(rows_hbm.at[src_v.at[pl.ds(j * G, G)]], bufs[b],
                       gsem[b])

    def gwait(j, b):
      pltpu.make_async_copy(rows_hbm.at[src_v.at[pl.ds(j * G, G)]], bufs[b],
                            gsem[b]).wait()

    def scat(j, b):
      pltpu.async_copy(bufs[b], acc.at[dst_v.at[pl.ds(j * G, G)]], ssem[b],
                       add=True)

    def swait(j, b):
      pltpu.make_async_copy(bufs[b], acc.at[dst_v.at[pl.ds(j * G, G)]],
                            ssem[b]).wait()

    def step(j, b, has_prev, has_next):
      gwait(j, b)
      scat(j, b)
      if has_prev:
        swait(j - 1, (b - 1) % NB)
      if has_next:
        gather(j + NB - 1, (b - 1) % NB)

    for h in range(NST):
      pltpu.sync_copy(src_hbm.at[wid, pl.ds(h * EPH, EPH)], src_v)
      pltpu.sync_copy(dst_hbm.at[wid, pl.ds(h * EPH, EPH)], dst_v)
      for j in range(NB - 1):
        gather(j, j)
      for i in range(NB):
        step(i, i, i > 0, True)

      def body(g, carry):
        for i in range(NB):
          step(g * NB + i, i, True, True)
        return carry

      lax.fori_loop(1, NFH // NB - 1, body, 0)
      for i in range(NB):
        j = NFH - NB + i
        step(j, i, True, j + NB - 1 < NFH)
      swait(NFH - 1, NB - 1)
    plsc.subcore_barrier()
    pltpu.sync_copy(acc.at[pl.ds(s * RPT, RPT)],
                    out_hbm.at[c, pl.ds(s * RPT, RPT)])

  return agg


_agg128 = _make_agg(DIN, jnp.float32)


def _make_deg():
  mesh = plsc.VectorSubcoreMesh(core_axis_name="c", subcore_axis_name="s")

  @functools.partial(
      pl.kernel,
      out_type=jax.ShapeDtypeStruct((NC, 1, NPAD), jnp.float32),
      mesh=mesh,
      scratch_types=[
          pltpu.VMEM((NF, G), jnp.int32),
          pltpu.VMEM((G,), jnp.float32),
          pltpu.VMEM((RPT,), jnp.float32),
          pltpu.VMEM_SHARED((NPAD,), jnp.float32),
      ],
  )
  def deg(dst_hbm, ones_hbm, zeros_hbm, out_hbm,
          dst_v, ones_v, zeros_v, acc):
    c = lax.axis_index("c")
    s = lax.axis_index("s")
    wid = c * NS + s
    pltpu.sync_copy(dst_hbm.at[wid], dst_v)
    pltpu.sync_copy(ones_hbm, ones_v)
    pltpu.sync_copy(zeros_hbm, zeros_v)
    pltpu.sync_copy(zeros_v, acc.at[pl.ds(s * RPT, RPT)])
    plsc.subcore_barrier()

    def body(j, carry):
      pltpu.sync_copy(ones_v, acc.at[dst_v.at[j]], add=True)
      return carry

    lax.fori_loop(0, NF, body, 0)
    plsc.subcore_barrier()
    pltpu.sync_copy(acc.at[pl.ds(s * RPT, RPT)],
                    out_hbm.at[c, 0, pl.ds(s * RPT, RPT)])

  return deg


_deg = _make_deg()

_B = 2048


def _dinv_of(degp_ref):
  dp = degp_ref[0, 0, :] + degp_ref[1, 0, :]
  return lax.rsqrt(dp + 1.0)[:, None]


def _tc1_body(degp_ref, x_ref, xs_ref):
  xs_ref[...] = x_ref[...] * _dinv_of(degp_ref)


def _tc2_body(degp_ref, aggp_ref, xs_ref, w1_ref, b1_ref, w2_ref, t_ref):
  dinv = _dinv_of(degp_ref)
  agg = (aggp_ref[0] + aggp_ref[1] + xs_ref[...]) * dinv
  h = jnp.dot(agg, w1_ref[...], preferred_element_type=jnp.float32,
              precision=lax.Precision.HIGHEST) + b1_ref[...]
  h = jnp.maximum(h, 0.0)
  t = jnp.dot(h, w2_ref[...], preferred_element_type=jnp.float32,
              precision=lax.Precision.HIGHEST) * dinv
  t_ref[...] = jnp.concatenate([t, jnp.zeros_like(t)], axis=1)


def _tc3_body(degp_ref, aggp_ref, t_ref, b2_ref, o_ref):
  agg = (aggp_ref[0] + aggp_ref[1] + t_ref[...])[:, :DOUT]
  o_ref[...] = agg * _dinv_of(degp_ref) + b2_ref[...]


def _degp_spec():
  return pl.BlockSpec((NC, 1, _B), lambda i: (0, 0, i))


def _rows_spec(d):
  return pl.BlockSpec((_B, d), lambda i: (i, 0))


def _aggp_spec(d):
  return pl.BlockSpec((NC, _B, d), lambda i: (0, i, 0))


def _full_spec(shape):
  return pl.BlockSpec(shape, lambda i: tuple(0 for _ in shape))


_tc1 = pl.pallas_call(
    _tc1_body,
    grid=(NPAD // _B,),
    in_specs=[_degp_spec(), _rows_spec(DIN)],
    out_specs=_rows_spec(DIN),
    out_shape=jax.ShapeDtypeStruct((NPAD, DIN), jnp.float32),
)

_tc2 = pl.pallas_call(
    _tc2_body,
    grid=(NPAD // _B,),
    in_specs=[_degp_spec(), _aggp_spec(DIN), _rows_spec(DIN),
              _full_spec((DIN, DHID)), _full_spec((1, DHID)),
              _full_spec((DHID, DOUT))],
    out_specs=_rows_spec(DIN),
    out_shape=jax.ShapeDtypeStruct((NPAD, DIN), jnp.float32),
)

_tc3 = pl.pallas_call(
    _tc3_body,
    grid=(NPAD // _B,),
    in_specs=[_degp_spec(), _aggp_spec(DIN), _rows_spec(DIN),
              _full_spec((1, DOUT))],
    out_specs=_rows_spec(DOUT),
    out_shape=jax.ShapeDtypeStruct((NPAD, DOUT), jnp.float32),
)


def kernel(x, edge_index, W1, b1, W2, b2):
  ei = edge_index.astype(jnp.int32)
  npad_e = EPAD - E
  pad_src = jnp.arange(npad_e, dtype=jnp.int32) % N
  pad_dst = N + jnp.arange(npad_e, dtype=jnp.int32) % (NPAD - N)
  src = jnp.concatenate([ei[0], pad_src]).reshape(NW, EPW)
  dst = jnp.concatenate([ei[1], pad_dst]).reshape(NW, EPW)
  zeros_in = jnp.zeros((RPT, DIN), jnp.float32)
  ones_1d = jnp.ones((G,), jnp.float32)
  zeros_1d = jnp.zeros((RPT,), jnp.float32)

  xp = jnp.pad(x, ((0, NPAD - N), (0, 0)))
  degp = _deg(dst.reshape(NW, NF, G), ones_1d, zeros_1d)
  xs = _tc1(degp, xp)
  aggp1 = _agg128(src, dst, xs, zeros_in)
  t = _tc2(degp, aggp1, xs, W1, b1.reshape(1, DHID), W2)
  aggp2 = _agg128(src, dst, t, zeros_in)
  return _tc3(degp, aggp2, t, b2.reshape(1, DOUT))[:N]
